# baseline (speedup 1.0000x reference)
import numpy as np

import jax
import jax.numpy as jnp
from jax.experimental import pallas as pl

F32 = jnp.float32
BF16 = jnp.bfloat16
_EPS = 1e-5

_T = 8
_NHF = 196
_NLF = 49
_NH = _T * _NHF
_NL = _T * _NLF
_NODES_F = 245
_TAIL = _T * _NODES_F - _NH


def _phase_col(f, p):
    i, j = divmod(p, 14)
    return ((i % 2 * 2 + j % 2) * _T + f) * _NLF + (i // 2) * 7 + (j // 2)


def _build_consts():
    p_h = np.zeros((_NH, _TAIL), np.float32)
    p_l = np.zeros((_NL, _TAIL), np.float32)
    for k in range(_TAIL):
        tau, q = divmod(k, _NLF)
        a, b = divmod(q, 7)
        for pi in (0, 1):
            for pj in (0, 1):
                n = _NHF * tau + (2 * a + pi) * 14 + (2 * b + pj)
                f, pos = divmod(n, _NODES_F)
                if pos < _NHF:
                    p_h[_phase_col(f, pos), k] += 1.0
                else:
                    p_l[f * _NLF + (pos - _NHF), k] += 1.0
    s_h = np.ones((1, _NH), np.float32)
    for f in range(_T):
        for p in range(_NHF):
            if _NODES_F * f + p >= _NH:
                s_h[0, _phase_col(f, p)] = 1.0 / 9.0
    s_l = np.ones((1, _NL), np.float32)
    s_l[0, 6 * _NLF:] = 1.0 / 9.0
    m_all = np.zeros((_TAIL, 4 * 2 * _NLF), np.float32)
    for c in range(4):
        pi, pj = c // 2, c % 2
        for f in (6, 7):
            for a in range(7):
                for b in range(7):
                    p = (2 * a + pi) * 14 + (2 * b + pj)
                    n = _NODES_F * f + p
                    if n >= _NH:
                        m_all[n - _NH, c * 98 + (f - 6) * _NLF + a * 7 + b] = 1.0
    return p_h, p_l, s_h, s_l, m_all


def _down_kernel(h_ref, wd_ref, xpre_ref, sum_ref, sq_ref):
    @pl.when(pl.program_id(0) == 0)
    def _init():
        sum_ref[...] = jnp.zeros_like(sum_ref)
        sq_ref[...] = jnp.zeros_like(sq_ref)

    wd = wd_ref[...].astype(BF16)
    for f in range(_T):
        x = jax.lax.dot_general(wd, h_ref[f].astype(BF16),
                                (((1,), (0,)), ((), ())),
                                preferred_element_type=F32)
        xpre_ref[f] = x.astype(BF16)
        sum_ref[...] += jnp.sum(x, axis=1, keepdims=True)
        sq_ref[...] += jnp.sum(x * x, axis=1, keepdims=True)


def _gcn_conv_kernel(xpre_ref, l_ref, sum1_ref, sq1_ref, g1_ref, b1_ref,
                     gw_ref, gb_ref, wc_ref, ph_ref, pl_ref, sh_ref,
                     sl_ref, mall_ref, bz_ref, y_ref, flo_ref,
                     sum_ref, sq_ref):
    mm = lambda a, b: jax.lax.dot_general(
        a, b, (((1,), (0,)), ((), ())), preferred_element_type=F32)
    n1 = 64.0 * _NHF
    mean1 = sum1_ref[...] * (1.0 / n1)
    var1 = sq1_ref[...] * (1.0 / n1) - mean1 * mean1
    s1 = g1_ref[...] * jax.lax.rsqrt(var1 + _EPS)
    t1 = b1_ref[...] - mean1 * s1
    x = jnp.maximum(xpre_ref[0].astype(F32) * s1 + t1, 0.0).astype(BF16)
    gw = gw_ref[...].astype(BF16)
    xw_h = mm(gw, x)
    xw_l = mm(gw, l_ref[0])
    xw_hb = xw_h.astype(BF16)
    xw_lb = xw_l.astype(BF16)
    p_agg = mm(xw_hb, ph_ref[...]) + mm(xw_lb, pl_ref[...])
    base_l = xw_l * sl_ref[...] + gb_ref[...] + bz_ref[...]
    add_l = jnp.concatenate(
        [jnp.zeros_like(base_l[:, :294]),
         p_agg[:, 98:147], p_agg[:, 343:392]], axis=1)
    flo_ref[0] = (base_l + (2.0 / 3.0) * add_l).astype(BF16)
    t_add = mm(p_agg.astype(BF16), mall_ref[...])
    f_ho = xw_h * sh_ref[...] + gb_ref[...]
    lane = jax.lax.broadcasted_iota(jnp.int32, (1, _NL), 1)
    mask_a = (lane % 49) >= 7
    mask_b = (lane % 7) != 0

    chunks = []
    for c in range(4):
        ch = f_ho[:, c * _NL:(c + 1) * _NL]
        chunks.append(jnp.concatenate(
            [ch[:, :294],
             ch[:, 294:] + (2.0 / 3.0) * t_add[:, c * 98:(c + 1) * 98]],
            axis=1).astype(BF16))
    c0, c1, c2c, c3 = chunks

    def rolled(chunk, k, mask):
        r = jnp.concatenate([chunk[:, _NL - k:], chunk[:, :_NL - k]], axis=1)
        return jnp.where(mask, r, jnp.zeros_like(r))

    taps = [
        rolled(c3, 8, jnp.logical_and(mask_a, mask_b)),
        rolled(c2c, 7, mask_a),
        rolled(c3, 7, mask_a),
        rolled(c1, 1, mask_b),
        c0,
        c1,
        rolled(c3, 1, mask_b),
        c2c,
        c3,
    ]
    xcat = jnp.concatenate(taps, axis=0)
    y = mm(wc_ref[...], xcat)
    y_ref[0] = y.astype(BF16)

    @pl.when(pl.program_id(0) == 0)
    def _init():
        sum_ref[...] = jnp.zeros_like(sum_ref)
        sq_ref[...] = jnp.zeros_like(sq_ref)

    sum_ref[...] += jnp.sum(y, axis=1, keepdims=True)
    sq_ref[...] += jnp.sum(y * y, axis=1, keepdims=True)


def _final_kernel(y_ref, flo_ref, sum2_ref, sq2_ref, g2_ref, b2_ref, o_ref):
    n2 = 64.0 * _NLF
    mean2 = sum2_ref[...] * (1.0 / n2)
    var2 = sq2_ref[...] * (1.0 / n2) - mean2 * mean2
    s2 = g2_ref[...] * jax.lax.rsqrt(var2 + _EPS)
    t2 = b2_ref[...] - mean2 * s2
    res = (jnp.maximum(y_ref[0].astype(F32) * s2 + t2, 0.0)
           + flo_ref[0].astype(F32))
    for f in range(_T):
        o_ref[0, f] = res[:, _NLF * f:_NLF * (f + 1)]


def kernel(featureH, featureL, batch, W_down, b_down, bn1_g, bn1_b,
           gcn_W, gcn_b, W_up, b_up, bn2_g, bn2_b):
    bt, c1 = featureH.shape[0], featureH.shape[1]
    c2 = featureL.shape[1]
    G = bt // _T

    h_r = featureH.reshape(bt, c1, _NHF)
    l_p = (featureL.astype(BF16).reshape(G, _T, c2, _NLF)
           .transpose(0, 2, 1, 3).reshape(G, c2, _NL))
    w_cat = W_up.astype(BF16).transpose(0, 2, 3, 1).reshape(c2, 9 * c2)

    p_h, p_l, s_h, s_l, m_all = _build_consts()
    p_h, p_l = jnp.asarray(p_h, BF16), jnp.asarray(p_l, BF16)
    s_h, s_l = jnp.asarray(s_h), jnp.asarray(s_l)
    m_all = jnp.asarray(m_all, BF16)

    xpre, sum1, sq1 = pl.pallas_call(
        _down_kernel,
        grid=(G,),
        in_specs=[
            pl.BlockSpec((_T, c1, _NHF), lambda i: (i, 0, 0)),
            pl.BlockSpec((c2, c1), lambda i: (0, 0)),
        ],
        out_specs=[
            pl.BlockSpec((_T, c2, _NHF), lambda i: (i, 0, 0)),
            pl.BlockSpec((c2, 1), lambda i: (0, 0)),
            pl.BlockSpec((c2, 1), lambda i: (0, 0)),
        ],
        out_shape=[
            jax.ShapeDtypeStruct((bt, c2, _NHF), BF16),
            jax.ShapeDtypeStruct((c2, 1), F32),
            jax.ShapeDtypeStruct((c2, 1), F32),
        ],
    )(h_r, W_down)

    bz = (jnp.asarray(batch) - 8).astype(F32).reshape(1, 1)

    xpre_p = (xpre.reshape(G, _T, c2, 7, 2, 7, 2)
              .transpose(0, 2, 4, 6, 1, 3, 5).reshape(G, c2, _NH))

    y, flo, sum2, sq2 = pl.pallas_call(
        _gcn_conv_kernel,
        grid=(G,),
        in_specs=[
            pl.BlockSpec((1, c2, _NH), lambda i: (i, 0, 0)),
            pl.BlockSpec((1, c2, _NL), lambda i: (i, 0, 0)),
            pl.BlockSpec((c2, 1), lambda i: (0, 0)),
            pl.BlockSpec((c2, 1), lambda i: (0, 0)),
            pl.BlockSpec((c2, 1), lambda i: (0, 0)),
            pl.BlockSpec((c2, 1), lambda i: (0, 0)),
            pl.BlockSpec((c2, c2), lambda i: (0, 0)),
            pl.BlockSpec((c2, 1), lambda i: (0, 0)),
            pl.BlockSpec((c2, 9 * c2), lambda i: (0, 0)),
            pl.BlockSpec((_NH, _TAIL), lambda i: (0, 0)),
            pl.BlockSpec((_NL, _TAIL), lambda i: (0, 0)),
            pl.BlockSpec((1, _NH), lambda i: (0, 0)),
            pl.BlockSpec((1, _NL), lambda i: (0, 0)),
            pl.BlockSpec((_TAIL, 4 * 98), lambda i: (0, 0)),
            pl.BlockSpec((1, 1), lambda i: (0, 0)),
        ],
        out_specs=[
            pl.BlockSpec((1, c2, _NL), lambda i: (i, 0, 0)),
            pl.BlockSpec((1, c2, _NL), lambda i: (i, 0, 0)),
            pl.BlockSpec((c2, 1), lambda i: (0, 0)),
            pl.BlockSpec((c2, 1), lambda i: (0, 0)),
        ],
        out_shape=[
            jax.ShapeDtypeStruct((G, c2, _NL), BF16),
            jax.ShapeDtypeStruct((G, c2, _NL), BF16),
            jax.ShapeDtypeStruct((c2, 1), F32),
            jax.ShapeDtypeStruct((c2, 1), F32),
        ],
    )(xpre_p, l_p, sum1, sq1, bn1_g[:, None], bn1_b[:, None],
      gcn_W, gcn_b[:, None], w_cat,
      p_h, p_l, s_h, s_l, m_all, bz)

    out = pl.pallas_call(
        _final_kernel,
        grid=(G,),
        in_specs=[
            pl.BlockSpec((1, c2, _NL), lambda i: (i, 0, 0)),
            pl.BlockSpec((1, c2, _NL), lambda i: (i, 0, 0)),
            pl.BlockSpec((c2, 1), lambda i: (0, 0)),
            pl.BlockSpec((c2, 1), lambda i: (0, 0)),
            pl.BlockSpec((c2, 1), lambda i: (0, 0)),
            pl.BlockSpec((c2, 1), lambda i: (0, 0)),
        ],
        out_specs=pl.BlockSpec((1, _T, c2, _NLF), lambda i: (i, 0, 0, 0)),
        out_shape=jax.ShapeDtypeStruct((G, _T, c2, _NLF), F32),
    )(y, flo, sum2, sq2, bn2_g[:, None], bn2_b[:, None])

    return out.reshape(bt, c2, 7, 7)


# speedup vs baseline: 1.2922x; 1.2922x over previous
import numpy as np

import jax
import jax.numpy as jnp
from jax.experimental import pallas as pl
from jax.experimental.pallas import tpu as pltpu

F32 = jnp.float32
BF16 = jnp.bfloat16
_EPS = 1e-5

_T = 8
_NHF = 196
_NLF = 49
_NH = _T * _NHF
_NL = _T * _NLF
_NODES_F = 245
_TAIL = _T * _NODES_F - _NH


def _phase_col(f, p):
    i, j = divmod(p, 14)
    return ((i % 2 * 2 + j % 2) * _T + f) * _NLF + (i // 2) * 7 + (j // 2)


def _build_consts():
    p_h = np.zeros((_NH, _TAIL), np.float32)
    p_l = np.zeros((_NL, _TAIL), np.float32)
    for k in range(_TAIL):
        tau, q = divmod(k, _NLF)
        a, b = divmod(q, 7)
        for pi in (0, 1):
            for pj in (0, 1):
                n = _NHF * tau + (2 * a + pi) * 14 + (2 * b + pj)
                f, pos = divmod(n, _NODES_F)
                if pos < _NHF:
                    p_h[_phase_col(f, pos), k] += 1.0
                else:
                    p_l[f * _NLF + (pos - _NHF), k] += 1.0
    s_h = np.ones((1, _NH), np.float32)
    for f in range(_T):
        for p in range(_NHF):
            if _NODES_F * f + p >= _NH:
                s_h[0, _phase_col(f, p)] = 1.0 / 9.0
    s_l = np.ones((1, _NL), np.float32)
    s_l[0, 6 * _NLF:] = 1.0 / 9.0
    m_all = np.zeros((_TAIL, 4 * 2 * _NLF), np.float32)
    for c in range(4):
        pi, pj = c // 2, c % 2
        for f in (6, 7):
            for a in range(7):
                for b in range(7):
                    p = (2 * a + pi) * 14 + (2 * b + pj)
                    n = _NODES_F * f + p
                    if n >= _NH:
                        m_all[n - _NH, c * 98 + (f - 6) * _NLF + a * 7 + b] = 1.0
    perm = np.zeros((_NHF, _NHF), np.float32)
    for p in range(_NHF):
        i, j = divmod(p, 14)
        perm[p, (i % 2 * 2 + j % 2) * _NLF + (i // 2) * 7 + (j // 2)] = 1.0
    return p_h, p_l, s_h, s_l, m_all, perm


def _mix_kernel(h_ref, l_ref, wd_ref, g1_ref, b1_ref, gw_ref, gb_ref,
                wc_ref, ph_ref, pl_ref, sh_ref, sl_ref, mall_ref,
                perm_ref, g2_ref, b2_ref, bz_ref, o_ref,
                xp_sc, y_sc, flo_sc, sum1_sc, sq1_sc, sum2_sc, sq2_sc):
    i = pl.program_id(0)
    phase = i // _T
    s = i % _T
    mm = lambda a, b: jax.lax.dot_general(
        a, b, (((1,), (0,)), ((), ())), preferred_element_type=F32)
    n_px = 64.0 * _NHF

    @pl.when(i == 0)
    def _init1():
        sum1_sc[...] = jnp.zeros_like(sum1_sc)
        sq1_sc[...] = jnp.zeros_like(sq1_sc)
        sum2_sc[...] = jnp.zeros_like(sum2_sc)
        sq2_sc[...] = jnp.zeros_like(sq2_sc)

    @pl.when(phase == 0)
    def _down():
        wd = wd_ref[...].astype(BF16)
        perm = perm_ref[...]
        ssum = sum1_sc[...]
        ssq = sq1_sc[...]
        xp = []
        for f in range(_T):
            x_f = mm(wd, h_ref[f].astype(BF16))
            ssum += jnp.sum(x_f, axis=1, keepdims=True)
            ssq += jnp.sum(x_f * x_f, axis=1, keepdims=True)
            xp.append(mm(x_f.astype(BF16), perm).astype(BF16))
        sum1_sc[...] = ssum
        sq1_sc[...] = ssq
        planes = [jnp.concatenate(
            [xp[f][:, _NLF * c:_NLF * (c + 1)] for f in range(_T)], axis=1)
            for c in range(4)]
        xp_sc[s] = jnp.concatenate(planes, axis=1)

    @pl.when(phase == 1)
    def _gcn_conv():
        mean1 = sum1_sc[...] * (1.0 / n_px)
        var1 = sq1_sc[...] * (1.0 / n_px) - mean1 * mean1
        s1 = g1_ref[...] * jax.lax.rsqrt(var1 + _EPS)
        t1 = b1_ref[...] - mean1 * s1
        x = jnp.maximum(xp_sc[s].astype(F32) * s1 + t1, 0.0).astype(BF16)
        gw = gw_ref[...].astype(BF16)
        xw_h = mm(gw, x)
        xw_l = mm(gw, l_ref[0])
        xw_hb = xw_h.astype(BF16)
        p_agg = mm(xw_hb, ph_ref[...]) + mm(xw_l.astype(BF16), pl_ref[...])
        two3 = 2.0 / 3.0
        base_l = xw_l * sl_ref[...] + gb_ref[...] + bz_ref[...]
        add_l = jnp.concatenate(
            [jnp.zeros_like(base_l[:, :294]),
             p_agg[:, 98:147], p_agg[:, 343:392]], axis=1)
        flo_sc[s] = (base_l + two3 * add_l).astype(BF16)

        t_add = mm(p_agg.astype(BF16), mall_ref[...])
        f_ho = xw_h * sh_ref[...] + gb_ref[...]
        lane = jax.lax.broadcasted_iota(jnp.int32, (1, _NL), 1)
        mask_a = (lane % 49) >= 7
        mask_b = (lane % 7) != 0

        chunks = []
        for c in range(4):
            ch = f_ho[:, c * _NL:(c + 1) * _NL]
            chunks.append(jnp.concatenate(
                [ch[:, :294],
                 ch[:, 294:] + two3 * t_add[:, c * 98:(c + 1) * 98]],
                axis=1).astype(BF16))
        c0, c1, c2c, c3 = chunks

        def rolled(chunk, k, mask):
            r = jnp.concatenate([chunk[:, _NL - k:], chunk[:, :_NL - k]],
                                axis=1)
            return jnp.where(mask, r, jnp.zeros_like(r))

        taps = [
            rolled(c3, 8, jnp.logical_and(mask_a, mask_b)),
            rolled(c2c, 7, mask_a),
            rolled(c3, 7, mask_a),
            rolled(c1, 1, mask_b),
            c0,
            c1,
            rolled(c3, 1, mask_b),
            c2c,
            c3,
        ]
        xcat = jnp.concatenate(taps, axis=0)
        y = mm(wc_ref[...], xcat)
        y_sc[s] = y.astype(BF16)
        sum2_sc[...] += jnp.sum(y, axis=1, keepdims=True)
        sq2_sc[...] += jnp.sum(y * y, axis=1, keepdims=True)

    @pl.when(phase == 2)
    def _final():
        n2 = 64.0 * _NLF
        mean2 = sum2_sc[...] * (1.0 / n2)
        var2 = sq2_sc[...] * (1.0 / n2) - mean2 * mean2
        s2 = g2_ref[...] * jax.lax.rsqrt(var2 + _EPS)
        t2 = b2_ref[...] - mean2 * s2
        res = (jnp.maximum(y_sc[s].astype(F32) * s2 + t2, 0.0)
               + flo_sc[s].astype(F32))
        for f in range(_T):
            o_ref[0, f] = res[:, _NLF * f:_NLF * (f + 1)]


def kernel(featureH, featureL, batch, W_down, b_down, bn1_g, bn1_b,
           gcn_W, gcn_b, W_up, b_up, bn2_g, bn2_b):
    bt, c1 = featureH.shape[0], featureH.shape[1]
    c2 = featureL.shape[1]
    G = bt // _T

    h_r = featureH.reshape(bt, c1, _NHF)
    l_p = (featureL.astype(BF16).reshape(G, _T, c2, _NLF)
           .transpose(0, 2, 1, 3).reshape(G, c2, _NL))
    w_cat = W_up.astype(BF16).transpose(0, 2, 3, 1).reshape(c2, 9 * c2)

    p_h, p_l, s_h, s_l, m_all, perm = _build_consts()
    p_h, p_l = jnp.asarray(p_h, BF16), jnp.asarray(p_l, BF16)
    s_h, s_l = jnp.asarray(s_h), jnp.asarray(s_l)
    m_all, perm = jnp.asarray(m_all, BF16), jnp.asarray(perm, BF16)

    bz = (jnp.asarray(batch) - 8).astype(F32).reshape(1, 1)

    cmap = lambda i: (0, 0)
    out = pl.pallas_call(
        _mix_kernel,
        grid=(3 * _T,),
        in_specs=[
            pl.BlockSpec((_T, c1, _NHF),
                         lambda i: (jnp.where(i < _T, i, 0), 0, 0)),
            pl.BlockSpec((1, c2, _NL),
                         lambda i: (jnp.where(i // _T == 1, i % _T, 0), 0, 0)),
            pl.BlockSpec((c2, c1), cmap),
            pl.BlockSpec((c2, 1), cmap),
            pl.BlockSpec((c2, 1), cmap),
            pl.BlockSpec((c2, c2), cmap),
            pl.BlockSpec((c2, 1), cmap),
            pl.BlockSpec((c2, 9 * c2), cmap),
            pl.BlockSpec((_NH, _TAIL), cmap),
            pl.BlockSpec((_NL, _TAIL), cmap),
            pl.BlockSpec((1, _NH), cmap),
            pl.BlockSpec((1, _NL), cmap),
            pl.BlockSpec((_TAIL, 4 * 98), cmap),
            pl.BlockSpec((_NHF, _NHF), cmap),
            pl.BlockSpec((c2, 1), cmap),
            pl.BlockSpec((c2, 1), cmap),
            pl.BlockSpec((1, 1), cmap),
        ],
        out_specs=pl.BlockSpec(
            (1, _T, c2, _NLF),
            lambda i: (jnp.where(i // _T == 2, i % _T, 0), 0, 0, 0)),
        out_shape=jax.ShapeDtypeStruct((G, _T, c2, _NLF), F32),
        scratch_shapes=[
            pltpu.VMEM((G, c2, _NH), BF16),
            pltpu.VMEM((G, c2, _NL), BF16),
            pltpu.VMEM((G, c2, _NL), BF16),
            pltpu.VMEM((c2, 1), F32),
            pltpu.VMEM((c2, 1), F32),
            pltpu.VMEM((c2, 1), F32),
            pltpu.VMEM((c2, 1), F32),
        ],
    )(h_r, l_p, W_down, bn1_g[:, None], bn1_b[:, None], gcn_W,
      gcn_b[:, None], w_cat, p_h, p_l, s_h, s_l, m_all, perm,
      bn2_g[:, None], bn2_b[:, None], bz)

    return out.reshape(bt, c2, 7, 7)
